# Initial kernel scaffold; baseline (speedup 1.0000x reference)
import jax
import jax.numpy as jnp
from jax.experimental import pallas as pl


def kernel(input, table):
    raise NotImplementedError("write your pallas kernel here")


# speedup vs baseline: 1.8701x; 1.8701x over previous
import functools

import jax
import jax.numpy as jnp
from jax import lax
from jax.experimental import pallas as pl
from jax.experimental.pallas import tpu as pltpu
from jax.experimental.pallas import tpu_sc as plsc

_NC = 2
_NS = 16
_NW = _NC * _NS

_C = 128
_NBUF = 8


def _make_sc_gather(nch: int, d: int):
    mesh = plsc.VectorSubcoreMesh(core_axis_name="c", subcore_axis_name="s")
    b_total = _NW * nch * _C
    ngr = nch // _NBUF

    @functools.partial(
        pl.kernel,
        mesh=mesh,
        out_type=jax.ShapeDtypeStruct((b_total, d), jnp.float32),
        scratch_types=(
            [
                pltpu.VMEM((nch, _C), jnp.int32),
                pltpu.VMEM((_NBUF, _C, d), jnp.float32),
            ]
            + [pltpu.SemaphoreType.DMA] * _NBUF
            + [pltpu.SemaphoreType.DMA] * _NBUF
        ),
        compiler_params=pltpu.CompilerParams(use_tc_tiling_on_sc=False),
    )
    def sc_gather(ids_hbm, table_hbm, out_hbm, idx_v, rows_v, *sems):
        gsems = sems[:_NBUF]
        ssems = sems[_NBUF:]
        wid = lax.axis_index("s") * _NC + lax.axis_index("c")
        base = wid * (nch * _C)
        pltpu.sync_copy(ids_hbm.at[wid], idx_v)

        def group(g, carry):
            gds = []
            for b in range(_NBUF):
                i = g * _NBUF + b
                gds.append(
                    pltpu.async_copy(
                        table_hbm.at[idx_v.at[i]], rows_v.at[b], gsems[b]
                    )
                )
            sds = []
            for b in range(_NBUF):
                i = g * _NBUF + b
                gds[b].wait()
                sds.append(
                    pltpu.async_copy(
                        rows_v.at[b],
                        out_hbm.at[pl.ds(base + i * _C, _C)],
                        ssems[b],
                    )
                )
            for b in range(_NBUF):
                sds[b].wait()
            return carry

        lax.fori_loop(0, ngr, group, 0)

    return sc_gather


def kernel(input, table):
    bsz, hist = input.shape
    d = table.shape[1]
    n = bsz * hist
    assert n % (_NW * _C * _NBUF) == 0
    nch = n // (_NW * _C)
    ids = input.astype(jnp.int32).reshape(_NW, nch, _C)
    out = _make_sc_gather(nch, d)(ids, table)
    return out.reshape(bsz, hist, d)
